# Initial kernel scaffold; baseline (speedup 1.0000x reference)
import functools

import jax
import jax.numpy as jnp
from jax import lax
from jax.experimental import pallas as pl
from jax.experimental.pallas import tpu as pltpu
from jax.experimental.pallas import tpu_sc as plsc

_N = 10000
_E = 320000
_IN = 128
_H1, _F1 = 8, 16
_HF = _H1 * _F1
_NC, _NS = 2, 16
_NW = _NC * _NS
_EPW = _E // _NW
_C = 80
_NCH = _EPW // _C
_RPT = _N // _NS
_TB = 2000
_STW = 144
_f32 = jnp.float32
_i32 = jnp.int32

_mesh = plsc.VectorSubcoreMesh(core_axis_name="c", subcore_axis_name="s",
                               num_cores=_NC)


def _leaky(x):
    return jnp.maximum(x, 0.0) + 0.2 * jnp.minimum(x, 0.0)


def _tc1_body(x_ref, w_ref, al_ref, ar_ref, stab_ref, el_ref, er_ref):
    feat = lax.dot_general(x_ref[...], w_ref[...], (((1,), (0,)), ((), ())),
                           precision=lax.Precision.HIGHEST)
    el = lax.dot_general(feat, al_ref[...], (((1,), (0,)), ((), ())),
                         precision=lax.Precision.HIGHEST)
    er = lax.dot_general(feat, ar_ref[...], (((1,), (0,)), ((), ())),
                         precision=lax.Precision.HIGHEST)
    z8 = jnp.zeros((_TB, 8), _f32)
    stab_ref[...] = jnp.concatenate([feat, el, z8], axis=1)
    el_ref[...] = jnp.concatenate([el, z8], axis=1)
    er_ref[...] = jnp.concatenate([er, z8], axis=1)


_tc1 = pl.pallas_call(
    _tc1_body,
    grid=(_N // _TB,),
    in_specs=[pl.BlockSpec((_TB, _IN), lambda i: (i, 0)),
              pl.BlockSpec((_IN, _HF), lambda i: (0, 0)),
              pl.BlockSpec((_HF, _H1), lambda i: (0, 0)),
              pl.BlockSpec((_HF, _H1), lambda i: (0, 0))],
    out_specs=[pl.BlockSpec((_TB, _STW), lambda i: (i, 0)),
               pl.BlockSpec((_TB, 16), lambda i: (i, 0)),
               pl.BlockSpec((_TB, 16), lambda i: (i, 0))],
    out_shape=[jax.ShapeDtypeStruct((_N, _STW), _f32),
               jax.ShapeDtypeStruct((_N, 16), _f32),
               jax.ShapeDtypeStruct((_N, 16), _f32)],
)


def _tc2_body(h_ref, w_ref, al_ref, ar_ref, t2_ref):
    f2 = lax.dot_general(h_ref[...], w_ref[...], (((1,), (0,)), ((), ())),
                         precision=lax.Precision.HIGHEST)
    al = al_ref[0, 0]
    ar = ar_ref[0, 0]
    z5 = jnp.zeros((_TB, 5), _f32)
    t2_ref[...] = jnp.concatenate([f2 * al, f2 * ar, f2, z5], axis=1)


_tc2 = pl.pallas_call(
    _tc2_body,
    grid=(_N // _TB,),
    in_specs=[pl.BlockSpec((_TB, _HF), lambda i: (i, 0)),
              pl.BlockSpec((_HF, 1), lambda i: (0, 0)),
              pl.BlockSpec((1, 1), lambda i: (0, 0)),
              pl.BlockSpec((1, 1), lambda i: (0, 0))],
    out_specs=pl.BlockSpec((_TB, 8), lambda i: (i, 0)),
    out_shape=jax.ShapeDtypeStruct((_N, 8), _f32),
)


def _worker_prologue(src_hbm, dst_hbm, srcv, dstv):
    c = lax.axis_index("c")
    s = lax.axis_index("s")
    w = c * _NS + s
    pltpu.sync_copy(src_hbm.at[w], srcv)
    pltpu.sync_copy(dst_hbm.at[w], dstv)
    return c, s


def _p1_body(el_hbm, er_hbm, src_hbm, dst_hbm, z_hbm, den_hbm,
             srcv, dstv, elb, erb, exb, shden):
    c, s = _worker_prologue(src_hbm, dst_hbm, srcv, dstv)
    rows = pl.ds(s * _RPT, _RPT)
    pltpu.sync_copy(z_hbm.at[rows], shden.at[rows])
    plsc.subcore_barrier()

    @pl.loop(0, _NCH)
    def _chunk(j):
        pltpu.sync_copy(el_hbm.at[srcv.at[j]], elb)
        pltpu.sync_copy(er_hbm.at[dstv.at[j]], erb)

        @pl.loop(0, _C)
        def _edge(i):
            exb[i] = jnp.exp(_leaky(elb[i] + erb[i]))

        pltpu.sync_copy(exb, shden.at[dstv.at[j]], add=True)

    plsc.subcore_barrier()
    pltpu.sync_copy(shden.at[rows], den_hbm.at[c, rows])


_p1 = pl.kernel(
    _p1_body,
    out_type=jax.ShapeDtypeStruct((_NC, _N, 16), _f32),
    mesh=_mesh,
    scratch_types=[
        pltpu.VMEM((_NCH, _C), _i32),
        pltpu.VMEM((_NCH, _C), _i32),
        pltpu.VMEM((_C, 16), _f32),
        pltpu.VMEM((_C, 16), _f32),
        pltpu.VMEM((_C, 16), _f32),
        pltpu.VMEM_SHARED((_N, 16), _f32),
    ],
)


def _p2_body(stab_hbm, er_hbm, den_hbm, src_hbm, dst_hbm, z_hbm, out_hbm,
             srcv, dstv, stb, erb, dnb, msgb, a16, shout):
    c, s = _worker_prologue(src_hbm, dst_hbm, srcv, dstv)
    rows = pl.ds(s * _RPT, _RPT)
    pltpu.sync_copy(z_hbm.at[rows], shout.at[rows])
    plsc.subcore_barrier()

    @pl.loop(0, _NCH)
    def _chunk(j):
        pltpu.sync_copy(stab_hbm.at[srcv.at[j]], stb)
        pltpu.sync_copy(er_hbm.at[dstv.at[j]], erb)
        pltpu.sync_copy(den_hbm.at[dstv.at[j]], dnb)

        @pl.loop(0, _C)
        def _edge(i):
            v = stb[i, pl.ds(_HF, 16)] + erb[i]
            ex = jnp.exp(_leaky(v))
            a16[...] = ex / (dnb[i] + 1e-9)
            for k in range(_H1):
                ak = a16[k]
                msgb[i, pl.ds(k * _F1, _F1)] = stb[i, pl.ds(k * _F1, _F1)] * ak

        pltpu.sync_copy(msgb, shout.at[dstv.at[j]], add=True)

    plsc.subcore_barrier()
    pltpu.sync_copy(shout.at[rows], out_hbm.at[c, rows])


_p2 = pl.kernel(
    _p2_body,
    out_type=jax.ShapeDtypeStruct((_NC, _N, _HF), _f32),
    mesh=_mesh,
    scratch_types=[
        pltpu.VMEM((_NCH, _C), _i32),
        pltpu.VMEM((_NCH, _C), _i32),
        pltpu.VMEM((_C, _STW), _f32),
        pltpu.VMEM((_C, 16), _f32),
        pltpu.VMEM((_C, 16), _f32),
        pltpu.VMEM((_C, _HF), _f32),
        pltpu.VMEM((16,), _f32),
        pltpu.VMEM_SHARED((_N, _HF), _f32),
    ],
)


def _p3_body(t2_hbm, src_hbm, dst_hbm, z_hbm, den_hbm,
             srcv, dstv, t2v, msgb, shden):
    c, s = _worker_prologue(src_hbm, dst_hbm, srcv, dstv)
    pltpu.sync_copy(t2_hbm, t2v)
    rows = pl.ds(s * _RPT, _RPT)
    pltpu.sync_copy(z_hbm.at[rows], shden.at[rows])
    pltpu.sync_copy(z_hbm.at[pl.ds(0, _C)], msgb)
    plsc.subcore_barrier()

    iota = lax.iota(_i32, 16)
    c0 = jnp.zeros((16,), _i32)
    c1 = c0 + 1

    @pl.loop(0, _NCH)
    def _chunk(j):
        @pl.loop(0, _C // 16)
        def _vec(v):
            sidx = srcv[j, pl.ds(v * 16, 16)]
            didx = dstv[j, pl.ds(v * 16, 16)]
            elv = plsc.load_gather(t2v, [sidx, c0])
            erv = plsc.load_gather(t2v, [didx, c1])
            ex = jnp.exp(_leaky(elv + erv))
            plsc.store_scatter(msgb, [v * 16 + iota, c0], ex)

        pltpu.sync_copy(msgb, shden.at[dstv.at[j]], add=True)

    plsc.subcore_barrier()
    pltpu.sync_copy(shden.at[rows], den_hbm.at[c, rows])


_p3 = pl.kernel(
    _p3_body,
    out_type=jax.ShapeDtypeStruct((_NC, _N, 16), _f32),
    mesh=_mesh,
    scratch_types=[
        pltpu.VMEM((_NCH, _C), _i32),
        pltpu.VMEM((_NCH, _C), _i32),
        pltpu.VMEM((_N, 8), _f32),
        pltpu.VMEM((_C, 16), _f32),
        pltpu.VMEM_SHARED((_N, 16), _f32),
    ],
)


def _p4_body(t2_hbm, d2_hbm, src_hbm, dst_hbm, z_hbm, out_hbm,
             srcv, dstv, t2v, d2v, msgb, shout):
    c, s = _worker_prologue(src_hbm, dst_hbm, srcv, dstv)
    pltpu.sync_copy(t2_hbm, t2v)
    pltpu.sync_copy(d2_hbm, d2v)
    rows = pl.ds(s * _RPT, _RPT)
    pltpu.sync_copy(z_hbm.at[rows], shout.at[rows])
    pltpu.sync_copy(z_hbm.at[pl.ds(0, _C)], msgb)
    plsc.subcore_barrier()

    iota = lax.iota(_i32, 16)
    c0 = jnp.zeros((16,), _i32)
    c1 = c0 + 1
    c2 = c0 + 2

    @pl.loop(0, _NCH)
    def _chunk(j):
        @pl.loop(0, _C // 16)
        def _vec(v):
            sidx = srcv[j, pl.ds(v * 16, 16)]
            didx = dstv[j, pl.ds(v * 16, 16)]
            elv = plsc.load_gather(t2v, [sidx, c0])
            erv = plsc.load_gather(t2v, [didx, c1])
            f2v = plsc.load_gather(t2v, [sidx, c2])
            dnv = plsc.load_gather(d2v, [didx])
            ex = jnp.exp(_leaky(elv + erv))
            val = f2v * ex / (dnv + 1e-9)
            plsc.store_scatter(msgb, [v * 16 + iota, c0], val)

        pltpu.sync_copy(msgb, shout.at[dstv.at[j]], add=True)

    plsc.subcore_barrier()
    pltpu.sync_copy(shout.at[rows], out_hbm.at[c, rows])


_p4 = pl.kernel(
    _p4_body,
    out_type=jax.ShapeDtypeStruct((_NC, _N, 16), _f32),
    mesh=_mesh,
    scratch_types=[
        pltpu.VMEM((_NCH, _C), _i32),
        pltpu.VMEM((_NCH, _C), _i32),
        pltpu.VMEM((_N, 8), _f32),
        pltpu.VMEM((_N,), _f32),
        pltpu.VMEM((_C, 16), _f32),
        pltpu.VMEM_SHARED((_N, 16), _f32),
    ],
)


@jax.jit
def kernel(inputs, edge_index, W1, attn_l1, attn_r1, bias1,
           W2, attn_l2, attn_r2, bias2):
    src = edge_index[0].reshape(_NW, _NCH, _C)
    dst = edge_index[1].reshape(_NW, _NCH, _C)

    eye = jnp.eye(_H1, dtype=_f32)
    AL = (attn_l1[:, :, None] * eye[:, None, :]).reshape(_HF, _H1)
    AR = (attn_r1[:, :, None] * eye[:, None, :]).reshape(_HF, _H1)

    z128 = jnp.zeros((_N, _HF), _f32)
    z16 = jnp.zeros((_N, 16), _f32)

    stab, eltab, ertab = _tc1(inputs, W1, AL, AR)

    den = _p1(eltab, ertab, src, dst, z16)
    dden = den[0, :, :8] + den[1, :, :8]
    dentab = jnp.concatenate([dden, jnp.zeros((_N, 8), _f32)], axis=1)

    out1 = _p2(stab, ertab, dentab, src, dst, z128)
    h = out1[0] + out1[1] + bias1[None, :]

    t2 = _tc2(h, W2, attn_l2, attn_r2)

    den2 = _p3(t2, src, dst, z16)
    d2 = den2[0, :, 0] + den2[1, :, 0]

    out2 = _p4(t2, d2, src, dst, z16)
    return out2[0, :, 0] + out2[1, :, 0] + bias2[0]


# speedup vs baseline: 38.6242x; 38.6242x over previous
import functools

import jax
import jax.numpy as jnp
from jax import lax
from jax.experimental import pallas as pl
from jax.experimental.pallas import tpu as pltpu
from jax.experimental.pallas import tpu_sc as plsc

_N = 10000
_E = 320000
_IN = 128
_H1, _F1 = 8, 16
_HF = _H1 * _F1
_NC, _NS = 2, 16
_NW = _NC * _NS
_EPW = _E // _NW
_C = 80
_NCH = _EPW // _C
_NP = 10240
_RPT = _NP // _NS
_TB = 2048
_STW = 144
_f32 = jnp.float32
_i32 = jnp.int32

_mesh = plsc.VectorSubcoreMesh(core_axis_name="c", subcore_axis_name="s",
                               num_cores=_NC)
_sc_params = pltpu.CompilerParams(use_tc_tiling_on_sc=False,
                                  needs_layout_passes=False)


def _leaky(x):
    return jnp.maximum(x, 0.0) + 0.2 * jnp.minimum(x, 0.0)


def _tc1_body(x_ref, w_ref, al_ref, ar_ref, stab_ref, el_ref, er_ref):
    feat = lax.dot_general(x_ref[...], w_ref[...], (((1,), (0,)), ((), ())),
                           precision=lax.Precision.HIGHEST)
    el = lax.dot_general(feat, al_ref[...], (((1,), (0,)), ((), ())),
                         precision=lax.Precision.HIGHEST)
    er = lax.dot_general(feat, ar_ref[...], (((1,), (0,)), ((), ())),
                         precision=lax.Precision.HIGHEST)
    z8 = jnp.zeros((_TB, 8), _f32)
    stab_ref[...] = jnp.concatenate([feat, el, z8], axis=1)
    el_ref[...] = jnp.concatenate([el, z8], axis=1)
    er_ref[...] = jnp.concatenate([er, z8], axis=1)


_tc1 = pl.pallas_call(
    _tc1_body,
    grid=(_NP // _TB,),
    in_specs=[pl.BlockSpec((_TB, _IN), lambda i: (i, 0)),
              pl.BlockSpec((_IN, _HF), lambda i: (0, 0)),
              pl.BlockSpec((_HF, _H1), lambda i: (0, 0)),
              pl.BlockSpec((_HF, _H1), lambda i: (0, 0))],
    out_specs=[pl.BlockSpec((_TB, _STW), lambda i: (i, 0)),
               pl.BlockSpec((_TB, 16), lambda i: (i, 0)),
               pl.BlockSpec((_TB, 16), lambda i: (i, 0))],
    out_shape=[jax.ShapeDtypeStruct((_NP, _STW), _f32),
               jax.ShapeDtypeStruct((_NP, 16), _f32),
               jax.ShapeDtypeStruct((_NP, 16), _f32)],
)


def _tc2_body(h_ref, w_ref, al_ref, ar_ref, t2_ref):
    f2 = lax.dot_general(h_ref[...], w_ref[...], (((1,), (0,)), ((), ())),
                         precision=lax.Precision.HIGHEST)
    al = al_ref[0, 0]
    ar = ar_ref[0, 0]
    z5 = jnp.zeros((_TB, 5), _f32)
    t2_ref[...] = jnp.concatenate([f2 * al, f2 * ar, f2, z5], axis=1)


_tc2 = pl.pallas_call(
    _tc2_body,
    grid=(_NP // _TB,),
    in_specs=[pl.BlockSpec((_TB, _HF), lambda i: (i, 0)),
              pl.BlockSpec((_HF, 1), lambda i: (0, 0)),
              pl.BlockSpec((1, 1), lambda i: (0, 0)),
              pl.BlockSpec((1, 1), lambda i: (0, 0))],
    out_specs=pl.BlockSpec((_TB, 8), lambda i: (i, 0)),
    out_shape=jax.ShapeDtypeStruct((_NP, 8), _f32),
)


def _worker_prologue(src_hbm, dst_hbm, srcv, dstv):
    c = lax.axis_index("c")
    s = lax.axis_index("s")
    w = c * _NS + s
    pltpu.sync_copy(src_hbm.at[w], srcv)
    pltpu.sync_copy(dst_hbm.at[w], dstv)
    return c, s


def _p1_body(el_hbm, er_hbm, src_hbm, dst_hbm, z_hbm, den_hbm,
             srcv, dstv, elb, erb, exb, shden):
    c, s = _worker_prologue(src_hbm, dst_hbm, srcv, dstv)
    rows = pl.ds(s * _RPT, _RPT)
    pltpu.sync_copy(z_hbm.at[rows], shden.at[rows])
    plsc.subcore_barrier()

    @pl.loop(0, _NCH)
    def _chunk(j):
        pltpu.sync_copy(el_hbm.at[srcv.at[j]], elb)
        pltpu.sync_copy(er_hbm.at[dstv.at[j]], erb)

        @pl.loop(0, _C)
        def _edge(i):
            exb[i] = jnp.exp(_leaky(elb[i] + erb[i]))

        pltpu.sync_copy(exb, shden.at[dstv.at[j]], add=True)

    plsc.subcore_barrier()
    pltpu.sync_copy(shden.at[rows], den_hbm.at[c, rows])


_p1 = pl.kernel(
    _p1_body,
    out_type=jax.ShapeDtypeStruct((_NC, _NP, 16), _f32),
    mesh=_mesh,
    compiler_params=_sc_params,
    scratch_types=[
        pltpu.VMEM((_NCH, _C), _i32),
        pltpu.VMEM((_NCH, _C), _i32),
        pltpu.VMEM((_C, 16), _f32),
        pltpu.VMEM((_C, 16), _f32),
        pltpu.VMEM((_C, 16), _f32),
        pltpu.VMEM_SHARED((_NP, 16), _f32),
    ],
)


def _p2_body(stab_hbm, er_hbm, den_hbm, src_hbm, dst_hbm, z_hbm, out_hbm,
             srcv, dstv, stb, erb, dnb, msgb, shout):
    c, s = _worker_prologue(src_hbm, dst_hbm, srcv, dstv)
    rows = pl.ds(s * _RPT, _RPT)
    pltpu.sync_copy(z_hbm.at[rows], shout.at[rows])
    plsc.subcore_barrier()

    @pl.loop(0, _NCH)
    def _chunk(j):
        pltpu.sync_copy(stab_hbm.at[srcv.at[j]], stb)
        pltpu.sync_copy(er_hbm.at[dstv.at[j]], erb)
        pltpu.sync_copy(den_hbm.at[dstv.at[j]], dnb)

        @pl.loop(0, _C)
        def _edge(i):
            v = stb[i, pl.ds(_HF, 16)] + erb[i]
            ex = jnp.exp(_leaky(v))
            alpha = ex / (dnb[i] + 1e-9)
            for k in range(_H1):
                msgb[i, pl.ds(k * _F1, _F1)] = stb[i, pl.ds(k * _F1, _F1)] * alpha[k]

        pltpu.sync_copy(msgb, shout.at[dstv.at[j]], add=True)

    plsc.subcore_barrier()
    pltpu.sync_copy(shout.at[rows], out_hbm.at[c, rows])


_p2 = pl.kernel(
    _p2_body,
    out_type=jax.ShapeDtypeStruct((_NC, _NP, _HF), _f32),
    mesh=_mesh,
    compiler_params=_sc_params,
    scratch_types=[
        pltpu.VMEM((_NCH, _C), _i32),
        pltpu.VMEM((_NCH, _C), _i32),
        pltpu.VMEM((_C, _STW), _f32),
        pltpu.VMEM((_C, 16), _f32),
        pltpu.VMEM((_C, 16), _f32),
        pltpu.VMEM((_C, _HF), _f32),
        pltpu.VMEM_SHARED((_NP, _HF), _f32),
    ],
)


def _p3_body(t2_hbm, src_hbm, dst_hbm, z_hbm, den_hbm,
             srcv, dstv, t2v, msgb, shden):
    c, s = _worker_prologue(src_hbm, dst_hbm, srcv, dstv)
    pltpu.sync_copy(t2_hbm, t2v)
    rows = pl.ds(s * _RPT, _RPT)
    pltpu.sync_copy(z_hbm.at[rows], shden.at[rows])
    pltpu.sync_copy(z_hbm.at[pl.ds(0, _C)], msgb)
    plsc.subcore_barrier()

    iota = lax.iota(_i32, 16)
    c0 = jnp.zeros((16,), _i32)
    c1 = c0 + 1

    @pl.loop(0, _NCH)
    def _chunk(j):
        @pl.loop(0, _C // 16)
        def _vec(v):
            sidx = srcv[j, pl.ds(v * 16, 16)]
            didx = dstv[j, pl.ds(v * 16, 16)]
            elv = plsc.load_gather(t2v, [sidx, c0])
            erv = plsc.load_gather(t2v, [didx, c1])
            ex = jnp.exp(_leaky(elv + erv))
            plsc.store_scatter(msgb, [v * 16 + iota, c0], ex)

        pltpu.sync_copy(msgb, shden.at[dstv.at[j]], add=True)

    plsc.subcore_barrier()
    pltpu.sync_copy(shden.at[rows], den_hbm.at[c, rows])


_p3 = pl.kernel(
    _p3_body,
    out_type=jax.ShapeDtypeStruct((_NC, _NP, 16), _f32),
    mesh=_mesh,
    compiler_params=_sc_params,
    scratch_types=[
        pltpu.VMEM((_NCH, _C), _i32),
        pltpu.VMEM((_NCH, _C), _i32),
        pltpu.VMEM((_NP, 8), _f32),
        pltpu.VMEM((_C, 16), _f32),
        pltpu.VMEM_SHARED((_NP, 16), _f32),
    ],
)


def _p4_body(t2_hbm, d2_hbm, src_hbm, dst_hbm, z_hbm, out_hbm,
             srcv, dstv, t2v, d2v, msgb, shout):
    c, s = _worker_prologue(src_hbm, dst_hbm, srcv, dstv)
    pltpu.sync_copy(t2_hbm, t2v)
    pltpu.sync_copy(d2_hbm, d2v)
    rows = pl.ds(s * _RPT, _RPT)
    pltpu.sync_copy(z_hbm.at[rows], shout.at[rows])
    pltpu.sync_copy(z_hbm.at[pl.ds(0, _C)], msgb)
    plsc.subcore_barrier()

    iota = lax.iota(_i32, 16)
    c0 = jnp.zeros((16,), _i32)
    c1 = c0 + 1
    c2 = c0 + 2

    @pl.loop(0, _NCH)
    def _chunk(j):
        @pl.loop(0, _C // 16)
        def _vec(v):
            sidx = srcv[j, pl.ds(v * 16, 16)]
            didx = dstv[j, pl.ds(v * 16, 16)]
            elv = plsc.load_gather(t2v, [sidx, c0])
            erv = plsc.load_gather(t2v, [didx, c1])
            f2v = plsc.load_gather(t2v, [sidx, c2])
            dnv = plsc.load_gather(d2v, [didx])
            ex = jnp.exp(_leaky(elv + erv))
            val = f2v * ex / (dnv + 1e-9)
            plsc.store_scatter(msgb, [v * 16 + iota, c0], val)

        pltpu.sync_copy(msgb, shout.at[dstv.at[j]], add=True)

    plsc.subcore_barrier()
    pltpu.sync_copy(shout.at[rows], out_hbm.at[c, rows])


_p4 = pl.kernel(
    _p4_body,
    out_type=jax.ShapeDtypeStruct((_NC, _NP, 16), _f32),
    mesh=_mesh,
    compiler_params=_sc_params,
    scratch_types=[
        pltpu.VMEM((_NCH, _C), _i32),
        pltpu.VMEM((_NCH, _C), _i32),
        pltpu.VMEM((_NP, 8), _f32),
        pltpu.VMEM((_NP,), _f32),
        pltpu.VMEM((_C, 16), _f32),
        pltpu.VMEM_SHARED((_NP, 16), _f32),
    ],
)


@jax.jit
def kernel(inputs, edge_index, W1, attn_l1, attn_r1, bias1,
           W2, attn_l2, attn_r2, bias2):
    src = edge_index[0].reshape(_NW, _NCH, _C)
    dst = edge_index[1].reshape(_NW, _NCH, _C)

    eye = jnp.eye(_H1, dtype=_f32)
    AL = (attn_l1[:, :, None] * eye[:, None, :]).reshape(_HF, _H1)
    AR = (attn_r1[:, :, None] * eye[:, None, :]).reshape(_HF, _H1)

    z128 = jnp.zeros((_NP, _HF), _f32)
    z16 = jnp.zeros((_NP, 16), _f32)

    xp = jnp.pad(inputs, ((0, _NP - _N), (0, 0)))
    stab, eltab, ertab = _tc1(xp, W1, AL, AR)

    den = _p1(eltab, ertab, src, dst, z16)
    dden = den[0, :, :8] + den[1, :, :8]
    dentab = jnp.concatenate([dden, jnp.zeros((_NP, 8), _f32)], axis=1)

    out1 = _p2(stab, ertab, dentab, src, dst, z128)
    h = out1[0] + out1[1] + bias1[None, :]

    t2 = _tc2(h, W2, attn_l2, attn_r2)

    den2 = _p3(t2, src, dst, z16)
    d2 = den2[0, :, 0] + den2[1, :, 0]

    out2 = _p4(t2, d2, src, dst, z16)
    return out2[0, :_N, 0] + out2[1, :_N, 0] + bias2[0]
